# baseline (speedup 1.0000x reference)
import functools

import jax
import jax.numpy as jnp
import numpy as np
from jax import lax
from jax.experimental import pallas as pl
from jax.experimental.pallas import tpu as pltpu
from jax.experimental.pallas import tpu_sc as plsc

B = 2
C = 128
V = 49152
DEG = 8
K = 3
CH = 1024
NCH = V // CH
ECH = DEG * CH
SCH = 2048
NSCH = V // SCH
RING = 4
VT = 512

_HI = np.int32(-65536)
_RND = np.int32(0x8000)


def _bf16_pair(bits0, bits1):
    lo = ((bits0 + _RND) >> 16) & 0xFFFF
    return lo | ((bits1 + _RND) & _HI)


def _sc_chebyshev(x, edges):
    info = plsc.get_sparse_core_info()
    nc, ns = info.num_cores, info.num_subcores
    nw = nc * ns
    cpw = C // nw
    mesh = plsc.VectorSubcoreMesh(core_axis_name="c", subcore_axis_name="s")

    @functools.partial(
        pl.kernel,
        mesh=mesh,
        out_type=(
            jax.ShapeDtypeStruct((C, V), jnp.int32),
            jax.ShapeDtypeStruct((C, V), jnp.int32),
        ),
        scratch_types=[
            pltpu.VMEM((V,), jnp.int32),
            pltpu.VMEM((RING * ECH,), jnp.int32),
            pltpu.VMEM((RING * CH,), jnp.int32),
            pltpu.VMEM((2 * 2 * SCH,), jnp.float32),
            [pltpu.SemaphoreType.DMA] * RING,
            [pltpu.SemaphoreType.DMA] * RING,
        ],
        compiler_params=pltpu.CompilerParams(needs_layout_passes=False),
    )
    def k(x_hbm, e_hbm, x1_hbm, z_hbm, x0p, ebuf, oring, stg, se, sy):
        wid = lax.axis_index("s") * nc + lax.axis_index("c")

        def stage_pack(c):
            def issue(sch, p, sem):
                off = sch * SCH
                pltpu.async_copy(x_hbm.at[0, c, pl.ds(off, SCH)],
                                 stg.at[pl.ds(p * 2 * SCH, SCH)], sem)
                pltpu.async_copy(x_hbm.at[1, c, pl.ds(off, SCH)],
                                 stg.at[pl.ds(p * 2 * SCH + SCH, SCH)], sem)

            def wait(p, sem):
                pltpu.make_async_copy(
                    x_hbm.at[0, c, pl.ds(0, 2 * SCH)],
                    stg.at[pl.ds(0, 2 * SCH)], sem).wait()

            issue(0, 0, se[0])
            issue(1, 1, se[1])

            def sbody(i, carry):
                for p in range(2):
                    sem = se[p]
                    sch = i * 2 + p
                    wait(p, sem)
                    pb = p * 2 * SCH

                    @plsc.parallel_loop(0, SCH // 16, unroll=4)
                    def pk(g):
                        b0 = lax.bitcast_convert_type(
                            stg[pl.ds(pb + g * 16, 16)], jnp.int32)
                        b1 = lax.bitcast_convert_type(
                            stg[pl.ds(pb + SCH + g * 16, 16)], jnp.int32)
                        x0p[pl.ds(sch * SCH + g * 16, 16)] = _bf16_pair(b0, b1)

                    @pl.when(sch + 2 < NSCH)
                    def _():
                        issue(sch + 2, p, sem)
                return carry

            lax.fori_loop(0, NSCH // 2, sbody, 0)

        def l_apply(c, src, dst_hbm):
            for p in range(RING):
                pltpu.async_copy(e_hbm.at[p], ebuf.at[pl.ds(p * ECH, ECH)],
                                 se[p])

            def chunk(i, ch, p):
                ebase = p * ECH
                ybase = p * CH
                pltpu.make_async_copy(
                    e_hbm.at[ch], ebuf.at[pl.ds(ebase, ECH)], se[p]).wait()

                @pl.when(i > 0)
                def _():
                    pltpu.make_async_copy(
                        oring.at[pl.ds(ybase, CH)],
                        dst_hbm.at[0, pl.ds(0, CH)], sy[p]).wait()

                def one_group(g2):
                    base = g2 * 16
                    p0 = []
                    p1 = []
                    for d in range(0, DEG, 2):
                        a0 = a1 = None
                        for dd in (d, d + 1):
                            we = ebuf[pl.ds(ebase + dd * CH + base, 16)]
                            col = we & 0xFFFF
                            val = lax.bitcast_convert_type(we & _HI,
                                                           jnp.float32)
                            wx = plsc.load_gather(src, [col])
                            m0 = val * lax.bitcast_convert_type(
                                wx << 16, jnp.float32)
                            m1 = val * lax.bitcast_convert_type(
                                wx & _HI, jnp.float32)
                            a0 = m0 if a0 is None else a0 + m0
                            a1 = m1 if a1 is None else a1 + m1
                        p0.append(a0)
                        p1.append(a1)
                    w = _bf16_pair(
                        lax.bitcast_convert_type(
                            (p0[0] + p0[1]) + (p0[2] + p0[3]), jnp.int32),
                        lax.bitcast_convert_type(
                            (p1[0] + p1[1]) + (p1[2] + p1[3]), jnp.int32))
                    oring[pl.ds(ybase + base, 16)] = w

                @plsc.parallel_loop(0, CH // 16, unroll=5)
                def grp(g):
                    one_group(g)
                pltpu.async_copy(oring.at[pl.ds(ybase, CH)],
                                 dst_hbm.at[c, pl.ds(ch * CH, CH)], sy[p])

                @pl.when(ch + RING < NCH)
                def _():
                    pltpu.async_copy(e_hbm.at[ch + RING],
                                     ebuf.at[pl.ds(ebase, ECH)], se[p])

            def bodyr(i, carry):
                for p in range(RING):
                    chunk(i, i * RING + p, p)
                return carry

            lax.fori_loop(0, NCH // RING, bodyr, 0)
            for p in range(RING):
                pltpu.make_async_copy(oring.at[pl.ds(0, CH)],
                                      dst_hbm.at[0, pl.ds(0, CH)],
                                      sy[p]).wait()

        def chan_body(i, carry):
            c = wid * cpw + i
            stage_pack(c)
            l_apply(c, x0p, x1_hbm)
            pltpu.sync_copy(x1_hbm.at[c], x0p)
            l_apply(c, x0p, z_hbm)
            return carry

        lax.fori_loop(0, cpw, chan_body, 0)

    return k(x, edges)


def _tc_einsum(x, x1p, zp, weight, bias2):
    def body(x_ref, x1_ref, z_ref, w_ref, b_ref, o_ref):
        b = pl.program_id(0)

        def unpack(wv):
            bits = jnp.where(b == 0, wv << 16, wv & _HI)
            return lax.bitcast_convert_type(bits, jnp.float32)

        w0 = w_ref[0]
        w1 = w_ref[1]
        w2 = w_ref[2]
        dn = (((0,), (0,)), ((), ()))
        acc = lax.dot_general(w0 - w2, x_ref[0], dn,
                              preferred_element_type=jnp.float32)
        acc = acc + lax.dot_general(w1, unpack(x1_ref[...]), dn,
                                    preferred_element_type=jnp.float32)
        acc = acc + 2.0 * lax.dot_general(w2, unpack(z_ref[...]), dn,
                                          preferred_element_type=jnp.float32)
        acc = acc + b_ref[...]
        o_ref[0] = jnp.maximum(acc, 0.0)

    bs3 = pl.BlockSpec((1, C, VT), lambda b, v: (b, 0, v))
    bs2 = pl.BlockSpec((C, VT), lambda b, v: (0, v))
    return pl.pallas_call(
        body,
        grid=(B, V // VT),
        in_specs=[bs3, bs2, bs2,
                  pl.BlockSpec((K, C, C), lambda b, v: (0, 0, 0)),
                  pl.BlockSpec((C, 1), lambda b, v: (0, 0))],
        out_specs=bs3,
        out_shape=jax.ShapeDtypeStruct((B, C, V), jnp.float32),
    )(x, x1p, zp, weight, bias2)


def kernel(x, lap_rows, lap_cols, lap_vals, weight, bias):
    del lap_rows
    vbits = lax.bitcast_convert_type(lap_vals, jnp.int32)
    packed = ((vbits + _RND) & _HI) | lap_cols
    edges = packed.reshape(NCH, CH, DEG).transpose(0, 2, 1).reshape(NCH, ECH)
    x1p, zp = _sc_chebyshev(x, edges)
    return _tc_einsum(x, x1p, zp, weight, bias2=bias.reshape(C, 1))


# speedup vs baseline: 1.2603x; 1.2603x over previous
import functools

import jax
import jax.numpy as jnp
import numpy as np
from jax import lax
from jax.experimental import pallas as pl
from jax.experimental.pallas import tpu as pltpu
from jax.experimental.pallas import tpu_sc as plsc

B = 2
C = 128
V = 49152
DEG = 8
K = 3
CH = 1024
NCH = V // CH
ECH = DEG * CH
SCH = 2048
NSCH = V // SCH
RING = 4
VT = 1024

_HI = np.int32(-65536)
_RND = np.int32(0x8000)


def _bf16_pair(bits0, bits1):
    lo = ((bits0 + _RND) >> 16) & 0xFFFF
    return lo | ((bits1 + _RND) & _HI)


def _sc_chebyshev(x, edges):
    info = plsc.get_sparse_core_info()
    nc, ns = info.num_cores, info.num_subcores
    nw = nc * ns
    cpw = C // nw
    mesh = plsc.VectorSubcoreMesh(core_axis_name="c", subcore_axis_name="s")

    @functools.partial(
        pl.kernel,
        mesh=mesh,
        out_type=(
            jax.ShapeDtypeStruct((C, V), jnp.int32),
            jax.ShapeDtypeStruct((C, V), jnp.int32),
        ),
        scratch_types=[
            pltpu.VMEM((V,), jnp.int32),
            pltpu.VMEM((RING * ECH,), jnp.int32),
            pltpu.VMEM((RING * CH,), jnp.int32),
            pltpu.VMEM((2 * 2 * SCH,), jnp.float32),
            [pltpu.SemaphoreType.DMA] * RING,
            [pltpu.SemaphoreType.DMA] * RING,
        ],
        compiler_params=pltpu.CompilerParams(needs_layout_passes=False),
    )
    def k(x_hbm, e_hbm, x1_hbm, z_hbm, x0p, ebuf, oring, stg, se, sy):
        wid = lax.axis_index("s") * nc + lax.axis_index("c")

        def stage_pack(c):
            def issue(sch, p, sem):
                off = sch * SCH
                pltpu.async_copy(x_hbm.at[0, c, pl.ds(off, SCH)],
                                 stg.at[pl.ds(p * 2 * SCH, SCH)], sem)
                pltpu.async_copy(x_hbm.at[1, c, pl.ds(off, SCH)],
                                 stg.at[pl.ds(p * 2 * SCH + SCH, SCH)], sem)

            def wait(p, sem):
                pltpu.make_async_copy(
                    x_hbm.at[0, c, pl.ds(0, 2 * SCH)],
                    stg.at[pl.ds(0, 2 * SCH)], sem).wait()

            issue(0, 0, se[0])
            issue(1, 1, se[1])

            def sbody(i, carry):
                for p in range(2):
                    sem = se[p]
                    sch = i * 2 + p
                    wait(p, sem)
                    pb = p * 2 * SCH

                    @plsc.parallel_loop(0, SCH // 16, unroll=4)
                    def pk(g):
                        b0 = lax.bitcast_convert_type(
                            stg[pl.ds(pb + g * 16, 16)], jnp.int32)
                        b1 = lax.bitcast_convert_type(
                            stg[pl.ds(pb + SCH + g * 16, 16)], jnp.int32)
                        x0p[pl.ds(sch * SCH + g * 16, 16)] = _bf16_pair(b0, b1)

                    @pl.when(sch + 2 < NSCH)
                    def _():
                        issue(sch + 2, p, sem)
                return carry

            lax.fori_loop(0, NSCH // 2, sbody, 0)

        def l_apply(c, src, dst_hbm):
            for p in range(RING):
                pltpu.async_copy(e_hbm.at[p], ebuf.at[pl.ds(p * ECH, ECH)],
                                 se[p])

            def chunk(i, ch, p):
                ebase = p * ECH
                ybase = p * CH
                pltpu.make_async_copy(
                    e_hbm.at[ch], ebuf.at[pl.ds(ebase, ECH)], se[p]).wait()

                @pl.when(i > 0)
                def _():
                    pltpu.make_async_copy(
                        oring.at[pl.ds(ybase, CH)],
                        dst_hbm.at[0, pl.ds(0, CH)], sy[p]).wait()

                def one_group(g2):
                    base = g2 * 16
                    p0 = []
                    p1 = []
                    for d in range(0, DEG, 2):
                        a0 = a1 = None
                        for dd in (d, d + 1):
                            we = ebuf[pl.ds(ebase + dd * CH + base, 16)]
                            col = we & 0xFFFF
                            val = lax.bitcast_convert_type(we & _HI,
                                                           jnp.float32)
                            wx = plsc.load_gather(src, [col])
                            m0 = val * lax.bitcast_convert_type(
                                wx << 16, jnp.float32)
                            m1 = val * lax.bitcast_convert_type(
                                wx & _HI, jnp.float32)
                            a0 = m0 if a0 is None else a0 + m0
                            a1 = m1 if a1 is None else a1 + m1
                        p0.append(a0)
                        p1.append(a1)
                    w = _bf16_pair(
                        lax.bitcast_convert_type(
                            (p0[0] + p0[1]) + (p0[2] + p0[3]), jnp.int32),
                        lax.bitcast_convert_type(
                            (p1[0] + p1[1]) + (p1[2] + p1[3]), jnp.int32))
                    oring[pl.ds(ybase + base, 16)] = w

                @plsc.parallel_loop(0, CH // 16, unroll=4)
                def grp(g):
                    one_group(g)
                pltpu.async_copy(oring.at[pl.ds(ybase, CH)],
                                 dst_hbm.at[c, pl.ds(ch * CH, CH)], sy[p])

                @pl.when(ch + RING < NCH)
                def _():
                    pltpu.async_copy(e_hbm.at[ch + RING],
                                     ebuf.at[pl.ds(ebase, ECH)], se[p])

            def bodyr(i, carry):
                for p in range(RING):
                    chunk(i, i * RING + p, p)
                return carry

            lax.fori_loop(0, NCH // RING, bodyr, 0)
            for p in range(RING):
                pltpu.make_async_copy(oring.at[pl.ds(0, CH)],
                                      dst_hbm.at[0, pl.ds(0, CH)],
                                      sy[p]).wait()

        def chan_body(i, carry):
            c = wid * cpw + i
            stage_pack(c)
            l_apply(c, x0p, x1_hbm)
            pltpu.sync_copy(x1_hbm.at[c], x0p)
            l_apply(c, x0p, z_hbm)
            return carry

        lax.fori_loop(0, cpw, chan_body, 0)

    return k(x, edges)


def _tc_einsum(x, x1p, zp, weight, bias2):
    def body(x_ref, x1_ref, z_ref, w_ref, b_ref, o_ref):
        b = pl.program_id(1)

        def unpack(wv):
            bits = jnp.where(b == 0, wv << 16, wv & _HI)
            return lax.bitcast_convert_type(bits, jnp.float32)

        w0 = w_ref[0]
        w1 = w_ref[1]
        w2 = w_ref[2]
        dn = (((0,), (0,)), ((), ()))
        acc = lax.dot_general(w0 - w2, x_ref[0], dn,
                              preferred_element_type=jnp.float32)
        acc = acc + lax.dot_general(w1, unpack(x1_ref[...]), dn,
                                    preferred_element_type=jnp.float32)
        acc = acc + 2.0 * lax.dot_general(w2, unpack(z_ref[...]), dn,
                                          preferred_element_type=jnp.float32)
        acc = acc + b_ref[...]
        o_ref[0] = jnp.maximum(acc, 0.0)

    bs3 = pl.BlockSpec((1, C, VT), lambda v, b: (b, 0, v))
    bs2 = pl.BlockSpec((C, VT), lambda v, b: (0, v))
    return pl.pallas_call(
        body,
        grid=(V // VT, B),
        in_specs=[bs3, bs2, bs2,
                  pl.BlockSpec((K, C, C), lambda v, b: (0, 0, 0)),
                  pl.BlockSpec((C, 1), lambda v, b: (0, 0))],
        out_specs=bs3,
        out_shape=jax.ShapeDtypeStruct((B, C, V), jnp.float32),
    )(x, x1p, zp, weight, bias2)


def kernel(x, lap_rows, lap_cols, lap_vals, weight, bias):
    del lap_rows
    vbits = lax.bitcast_convert_type(lap_vals, jnp.int32)
    packed = ((vbits + _RND) & _HI) | lap_cols
    edges = packed.reshape(NCH, CH, DEG).transpose(0, 2, 1).reshape(NCH, ECH)
    x1p, zp = _sc_chebyshev(x, edges)
    return _tc_einsum(x, x1p, zp, weight, bias2=bias.reshape(C, 1))


# speedup vs baseline: 1.3309x; 1.0561x over previous
import functools

import jax
import jax.numpy as jnp
import numpy as np
from jax import lax
from jax.experimental import pallas as pl
from jax.experimental.pallas import tpu as pltpu
from jax.experimental.pallas import tpu_sc as plsc

B = 2
C = 128
V = 49152
DEG = 8
K = 3
CH = 1024
NCH = V // CH
ECH = DEG * CH
SCH = 2048
NSCH = V // SCH
RING = 4
VT = 2048

_HI = np.int32(-65536)
_RND = np.int32(0x8000)


def _bf16_pair(bits0, bits1):
    lo = ((bits0 + _RND) >> 16) & 0xFFFF
    return lo | ((bits1 + _RND) & _HI)


def _sc_chebyshev(x, edges):
    info = plsc.get_sparse_core_info()
    nc, ns = info.num_cores, info.num_subcores
    nw = nc * ns
    cpw = C // nw
    mesh = plsc.VectorSubcoreMesh(core_axis_name="c", subcore_axis_name="s")

    @functools.partial(
        pl.kernel,
        mesh=mesh,
        out_type=(
            jax.ShapeDtypeStruct((C, V), jnp.int32),
            jax.ShapeDtypeStruct((C, V), jnp.int32),
        ),
        scratch_types=[
            pltpu.VMEM((V,), jnp.int32),
            pltpu.VMEM((RING * ECH,), jnp.int32),
            pltpu.VMEM((RING * CH,), jnp.int32),
            pltpu.VMEM((2 * 2 * SCH,), jnp.float32),
            [pltpu.SemaphoreType.DMA] * RING,
            [pltpu.SemaphoreType.DMA] * RING,
        ],
        compiler_params=pltpu.CompilerParams(needs_layout_passes=False),
    )
    def k(x_hbm, e_hbm, x1_hbm, z_hbm, x0p, ebuf, oring, stg, se, sy):
        wid = lax.axis_index("s") * nc + lax.axis_index("c")

        def stage_pack(c):
            def issue(sch, p, sem):
                off = sch * SCH
                pltpu.async_copy(x_hbm.at[0, c, pl.ds(off, SCH)],
                                 stg.at[pl.ds(p * 2 * SCH, SCH)], sem)
                pltpu.async_copy(x_hbm.at[1, c, pl.ds(off, SCH)],
                                 stg.at[pl.ds(p * 2 * SCH + SCH, SCH)], sem)

            def wait(p, sem):
                pltpu.make_async_copy(
                    x_hbm.at[0, c, pl.ds(0, 2 * SCH)],
                    stg.at[pl.ds(0, 2 * SCH)], sem).wait()

            issue(0, 0, se[0])
            issue(1, 1, se[1])

            def sbody(i, carry):
                for p in range(2):
                    sem = se[p]
                    sch = i * 2 + p
                    wait(p, sem)
                    pb = p * 2 * SCH

                    @plsc.parallel_loop(0, SCH // 16, unroll=4)
                    def pk(g):
                        b0 = lax.bitcast_convert_type(
                            stg[pl.ds(pb + g * 16, 16)], jnp.int32)
                        b1 = lax.bitcast_convert_type(
                            stg[pl.ds(pb + SCH + g * 16, 16)], jnp.int32)
                        x0p[pl.ds(sch * SCH + g * 16, 16)] = _bf16_pair(b0, b1)

                    @pl.when(sch + 2 < NSCH)
                    def _():
                        issue(sch + 2, p, sem)
                return carry

            lax.fori_loop(0, NSCH // 2, sbody, 0)

        def l_apply(c, src, dst_hbm):
            for p in range(RING):
                pltpu.async_copy(e_hbm.at[p], ebuf.at[pl.ds(p * ECH, ECH)],
                                 se[p])

            def chunk(i, ch, p):
                ebase = p * ECH
                ybase = p * CH
                pltpu.make_async_copy(
                    e_hbm.at[ch], ebuf.at[pl.ds(ebase, ECH)], se[p]).wait()

                @pl.when(i > 0)
                def _():
                    pltpu.make_async_copy(
                        oring.at[pl.ds(ybase, CH)],
                        dst_hbm.at[0, pl.ds(0, CH)], sy[p]).wait()

                def one_group(g2):
                    base = g2 * 16
                    p0 = []
                    p1 = []
                    for d in range(0, DEG, 2):
                        a0 = a1 = None
                        for dd in (d, d + 1):
                            we = ebuf[pl.ds(ebase + dd * CH + base, 16)]
                            col = we & 0xFFFF
                            val = lax.bitcast_convert_type(we & _HI,
                                                           jnp.float32)
                            wx = plsc.load_gather(src, [col])
                            m0 = val * lax.bitcast_convert_type(
                                wx << 16, jnp.float32)
                            m1 = val * lax.bitcast_convert_type(
                                wx & _HI, jnp.float32)
                            a0 = m0 if a0 is None else a0 + m0
                            a1 = m1 if a1 is None else a1 + m1
                        p0.append(a0)
                        p1.append(a1)
                    w = _bf16_pair(
                        lax.bitcast_convert_type(
                            (p0[0] + p0[1]) + (p0[2] + p0[3]), jnp.int32),
                        lax.bitcast_convert_type(
                            (p1[0] + p1[1]) + (p1[2] + p1[3]), jnp.int32))
                    oring[pl.ds(ybase + base, 16)] = w

                @plsc.parallel_loop(0, CH // 16, unroll=4)
                def grp(g):
                    one_group(g)
                pltpu.async_copy(oring.at[pl.ds(ybase, CH)],
                                 dst_hbm.at[c, pl.ds(ch * CH, CH)], sy[p])

                @pl.when(ch + RING < NCH)
                def _():
                    pltpu.async_copy(e_hbm.at[ch + RING],
                                     ebuf.at[pl.ds(ebase, ECH)], se[p])

            def bodyr(i, carry):
                for p in range(RING):
                    chunk(i, i * RING + p, p)
                return carry

            lax.fori_loop(0, NCH // RING, bodyr, 0)
            for p in range(RING):
                pltpu.make_async_copy(oring.at[pl.ds(0, CH)],
                                      dst_hbm.at[0, pl.ds(0, CH)],
                                      sy[p]).wait()

        def chan_body(i, carry):
            c = wid * cpw + i
            stage_pack(c)
            l_apply(c, x0p, x1_hbm)
            pltpu.sync_copy(x1_hbm.at[c], x0p)
            l_apply(c, x0p, z_hbm)
            return carry

        lax.fori_loop(0, cpw, chan_body, 0)

    return k(x, edges)


def _tc_einsum(x, x1p, zp, weight, bias2):
    def body(x_ref, x1_ref, z_ref, w_ref, b_ref, o_ref):
        b = pl.program_id(1)

        def unpack(wv):
            bits = jnp.where(b == 0, wv << 16, wv & _HI)
            return lax.bitcast_convert_type(bits, jnp.float32)

        w0 = w_ref[0]
        w1 = w_ref[1]
        w2 = w_ref[2]
        dn = (((0,), (0,)), ((), ()))
        acc = lax.dot_general(w0 - w2, x_ref[0], dn,
                              preferred_element_type=jnp.float32)
        acc = acc + lax.dot_general(w1, unpack(x1_ref[...]), dn,
                                    preferred_element_type=jnp.float32)
        acc = acc + 2.0 * lax.dot_general(w2, unpack(z_ref[...]), dn,
                                          preferred_element_type=jnp.float32)
        acc = acc + b_ref[...]
        o_ref[0] = jnp.maximum(acc, 0.0)

    bs3 = pl.BlockSpec((1, C, VT), lambda v, b: (b, 0, v))
    bs2 = pl.BlockSpec((C, VT), lambda v, b: (0, v))
    return pl.pallas_call(
        body,
        grid=(V // VT, B),
        in_specs=[bs3, bs2, bs2,
                  pl.BlockSpec((K, C, C), lambda v, b: (0, 0, 0)),
                  pl.BlockSpec((C, 1), lambda v, b: (0, 0))],
        out_specs=bs3,
        out_shape=jax.ShapeDtypeStruct((B, C, V), jnp.float32),
    )(x, x1p, zp, weight, bias2)


def kernel(x, lap_rows, lap_cols, lap_vals, weight, bias):
    del lap_rows
    vbits = lax.bitcast_convert_type(lap_vals, jnp.int32)
    packed = ((vbits + _RND) & _HI) | lap_cols
    edges = packed.reshape(NCH, CH, DEG).transpose(0, 2, 1).reshape(NCH, ECH)
    x1p, zp = _sc_chebyshev(x, edges)
    return _tc_einsum(x, x1p, zp, weight, bias2=bias.reshape(C, 1))


# speedup vs baseline: 1.3554x; 1.0184x over previous
import functools

import jax
import jax.numpy as jnp
import numpy as np
from jax import lax
from jax.experimental import pallas as pl
from jax.experimental.pallas import tpu as pltpu
from jax.experimental.pallas import tpu_sc as plsc

B = 2
C = 128
V = 49152
DEG = 8
K = 3
CH = 1024
NCH = V // CH
ECH = DEG * CH
SCH = 2048
NSCH = V // SCH
RING = 4
VT = 3072

_HI = np.int32(-65536)
_RND = np.int32(0x8000)


def _bf16_pair(bits0, bits1):
    lo = ((bits0 + _RND) >> 16) & 0xFFFF
    return lo | ((bits1 + _RND) & _HI)


def _sc_chebyshev(x, edges):
    info = plsc.get_sparse_core_info()
    nc, ns = info.num_cores, info.num_subcores
    nw = nc * ns
    cpw = C // nw
    mesh = plsc.VectorSubcoreMesh(core_axis_name="c", subcore_axis_name="s")

    @functools.partial(
        pl.kernel,
        mesh=mesh,
        out_type=(
            jax.ShapeDtypeStruct((C, V), jnp.int32),
            jax.ShapeDtypeStruct((C, V), jnp.int32),
        ),
        scratch_types=[
            pltpu.VMEM((V,), jnp.int32),
            pltpu.VMEM((RING * ECH,), jnp.int32),
            pltpu.VMEM((RING * CH,), jnp.int32),
            pltpu.VMEM((2 * 2 * SCH,), jnp.float32),
            [pltpu.SemaphoreType.DMA] * RING,
            [pltpu.SemaphoreType.DMA] * RING,
        ],
        compiler_params=pltpu.CompilerParams(needs_layout_passes=False),
    )
    def k(x_hbm, e_hbm, x1_hbm, z_hbm, x0p, ebuf, oring, stg, se, sy):
        wid = lax.axis_index("s") * nc + lax.axis_index("c")

        def stage_pack(c):
            def issue(sch, p, sem):
                off = sch * SCH
                pltpu.async_copy(x_hbm.at[0, c, pl.ds(off, SCH)],
                                 stg.at[pl.ds(p * 2 * SCH, SCH)], sem)
                pltpu.async_copy(x_hbm.at[1, c, pl.ds(off, SCH)],
                                 stg.at[pl.ds(p * 2 * SCH + SCH, SCH)], sem)

            def wait(p, sem):
                pltpu.make_async_copy(
                    x_hbm.at[0, c, pl.ds(0, 2 * SCH)],
                    stg.at[pl.ds(0, 2 * SCH)], sem).wait()

            issue(0, 0, se[0])
            issue(1, 1, se[1])

            def sbody(i, carry):
                for p in range(2):
                    sem = se[p]
                    sch = i * 2 + p
                    wait(p, sem)
                    pb = p * 2 * SCH

                    @plsc.parallel_loop(0, SCH // 16, unroll=4)
                    def pk(g):
                        b0 = lax.bitcast_convert_type(
                            stg[pl.ds(pb + g * 16, 16)], jnp.int32)
                        b1 = lax.bitcast_convert_type(
                            stg[pl.ds(pb + SCH + g * 16, 16)], jnp.int32)
                        x0p[pl.ds(sch * SCH + g * 16, 16)] = _bf16_pair(b0, b1)

                    @pl.when(sch + 2 < NSCH)
                    def _():
                        issue(sch + 2, p, sem)
                return carry

            lax.fori_loop(0, NSCH // 2, sbody, 0)

        def l_apply(c, src, dst_hbm):
            for p in range(RING):
                pltpu.async_copy(e_hbm.at[p], ebuf.at[pl.ds(p * ECH, ECH)],
                                 se[p])

            def chunk(i, ch, p):
                ebase = p * ECH
                ybase = p * CH
                pltpu.make_async_copy(
                    e_hbm.at[ch], ebuf.at[pl.ds(ebase, ECH)], se[p]).wait()

                @pl.when(i > 0)
                def _():
                    pltpu.make_async_copy(
                        oring.at[pl.ds(ybase, CH)],
                        dst_hbm.at[0, pl.ds(0, CH)], sy[p]).wait()

                def one_group(g2):
                    base = g2 * 16
                    p0 = []
                    p1 = []
                    for d in range(0, DEG, 2):
                        a0 = a1 = None
                        for dd in (d, d + 1):
                            we = ebuf[pl.ds(ebase + dd * CH + base, 16)]
                            col = we & 0xFFFF
                            val = lax.bitcast_convert_type(we & _HI,
                                                           jnp.float32)
                            wx = plsc.load_gather(src, [col])
                            m0 = val * lax.bitcast_convert_type(
                                wx << 16, jnp.float32)
                            m1 = val * lax.bitcast_convert_type(
                                wx & _HI, jnp.float32)
                            a0 = m0 if a0 is None else a0 + m0
                            a1 = m1 if a1 is None else a1 + m1
                        p0.append(a0)
                        p1.append(a1)
                    w = _bf16_pair(
                        lax.bitcast_convert_type(
                            (p0[0] + p0[1]) + (p0[2] + p0[3]), jnp.int32),
                        lax.bitcast_convert_type(
                            (p1[0] + p1[1]) + (p1[2] + p1[3]), jnp.int32))
                    oring[pl.ds(ybase + base, 16)] = w

                @plsc.parallel_loop(0, CH // 16, unroll=4)
                def grp(g):
                    one_group(g)
                pltpu.async_copy(oring.at[pl.ds(ybase, CH)],
                                 dst_hbm.at[c, pl.ds(ch * CH, CH)], sy[p])

                @pl.when(ch + RING < NCH)
                def _():
                    pltpu.async_copy(e_hbm.at[ch + RING],
                                     ebuf.at[pl.ds(ebase, ECH)], se[p])

            def bodyr(i, carry):
                for p in range(RING):
                    chunk(i, i * RING + p, p)
                return carry

            lax.fori_loop(0, NCH // RING, bodyr, 0)
            for p in range(RING):
                pltpu.make_async_copy(oring.at[pl.ds(0, CH)],
                                      dst_hbm.at[0, pl.ds(0, CH)],
                                      sy[p]).wait()

        def chan_body(i, carry):
            c = wid * cpw + i
            stage_pack(c)
            l_apply(c, x0p, x1_hbm)
            pltpu.sync_copy(x1_hbm.at[c], x0p)
            l_apply(c, x0p, z_hbm)
            return carry

        lax.fori_loop(0, cpw, chan_body, 0)

    return k(x, edges)


def _tc_einsum(x, x1p, zp, weight, bias2):
    def body(x_ref, x1_ref, z_ref, w_ref, b_ref, o_ref):
        b = pl.program_id(1)

        def unpack(wv):
            bits = jnp.where(b == 0, wv << 16, wv & _HI)
            return lax.bitcast_convert_type(bits, jnp.float32)

        w0 = w_ref[0]
        w1 = w_ref[1]
        w2 = w_ref[2]
        dn = (((0,), (0,)), ((), ()))
        acc = lax.dot_general(w0 - w2, x_ref[0], dn,
                              preferred_element_type=jnp.float32)
        acc = acc + lax.dot_general(w1, unpack(x1_ref[...]), dn,
                                    preferred_element_type=jnp.float32)
        acc = acc + 2.0 * lax.dot_general(w2, unpack(z_ref[...]), dn,
                                          preferred_element_type=jnp.float32)
        acc = acc + b_ref[...]
        o_ref[0] = jnp.maximum(acc, 0.0)

    bs3 = pl.BlockSpec((1, C, VT), lambda v, b: (b, 0, v))
    bs2 = pl.BlockSpec((C, VT), lambda v, b: (0, v))
    return pl.pallas_call(
        body,
        grid=(V // VT, B),
        in_specs=[bs3, bs2, bs2,
                  pl.BlockSpec((K, C, C), lambda v, b: (0, 0, 0)),
                  pl.BlockSpec((C, 1), lambda v, b: (0, 0))],
        out_specs=bs3,
        out_shape=jax.ShapeDtypeStruct((B, C, V), jnp.float32),
    )(x, x1p, zp, weight, bias2)


def kernel(x, lap_rows, lap_cols, lap_vals, weight, bias):
    del lap_rows
    vbits = lax.bitcast_convert_type(lap_vals, jnp.int32)
    packed = ((vbits + _RND) & _HI) | lap_cols
    edges = packed.reshape(NCH, CH, DEG).transpose(0, 2, 1).reshape(NCH, ECH)
    x1p, zp = _sc_chebyshev(x, edges)
    return _tc_einsum(x, x1p, zp, weight, bias2=bias.reshape(C, 1))
